# Initial kernel scaffold; baseline (speedup 1.0000x reference)
import jax
import jax.numpy as jnp
from jax.experimental import pallas as pl


def kernel(x, edge_index, W, att_src, att_dst, bias):
    raise NotImplementedError("write your pallas kernel here")


# speedup vs baseline: 13.4449x; 13.4449x over previous
import functools

import jax
import jax.numpy as jnp
from jax import lax
from jax.experimental import pallas as pl
from jax.experimental.pallas import tpu as pltpu
from jax.experimental.pallas import tpu_sc as plsc

N = 10000
D = 256
H = 4
C = 256
HC = H * C
NCHUNK = 8
FC = HC // NCHUNK
N_PAD = 10240
E_PAD = 196608
BN = 512

NCORES = 2
NSUB = 16

B1_BATCHES = E_PAD // (NCORES * NSUB) // 128
B1_EDGES = B1_BATCHES * 128
C_BATCHES = E_PAD // NSUB // 128
C_EDGES = C_BATCHES * 128
ROWS_PER_TILE = N_PAD // NSUB

_i32 = jnp.int32
_f32 = jnp.float32


def _iota16():
    return lax.iota(_i32, 16)


def _mm_body(x_ref, w_ref, out_ref):
    out_ref[...] = jnp.dot(
        x_ref[...], w_ref[0], preferred_element_type=_f32
    )[None]


def _matmul_chunks(x_pad, wc):
    grid = (N_PAD // BN, NCHUNK)
    return pl.pallas_call(
        _mm_body,
        grid=grid,
        in_specs=[
            pl.BlockSpec((BN, D), lambda i, c: (i, 0)),
            pl.BlockSpec((1, D, FC), lambda i, c: (c, 0, 0)),
        ],
        out_specs=pl.BlockSpec((1, BN, FC), lambda i, c: (c, i, 0)),
        out_shape=jax.ShapeDtypeStruct((NCHUNK, N_PAD, FC), _f32),
    )(x_pad, wc)


def _logits_body(x_ref, a_ref, out_ref, mx_ref):
    i = pl.program_id(0)
    a = jnp.dot(x_ref[...], a_ref[...], preferred_element_type=_f32)
    out_ref[...] = a
    bm = jnp.max(a, axis=0, keepdims=True)

    @pl.when(i == 0)
    def _():
        mx_ref[...] = jnp.full((1, 128), -3e38, _f32)

    mx_ref[...] = jnp.maximum(mx_ref[...], bm)


def _logits(x_pad, a8p):
    return pl.pallas_call(
        _logits_body,
        grid=(N_PAD // BN,),
        in_specs=[
            pl.BlockSpec((BN, D), lambda i: (i, 0)),
            pl.BlockSpec((D, 128), lambda i: (0, 0)),
        ],
        out_specs=[
            pl.BlockSpec((BN, 128), lambda i: (i, 0)),
            pl.BlockSpec((1, 128), lambda i: (0, 0)),
        ],
        out_shape=[
            jax.ShapeDtypeStruct((N_PAD, 128), _f32),
            jax.ShapeDtypeStruct((1, 128), _f32),
        ],
    )(x_pad, a8p)


def _edge_body(src1d, dst1d, asdf_hbm, k64_hbm,
               p_out, denom2,
               asd_sh, den_sh,
               sidx1, didx1, pbuf, kv, zv, gsidx, gdidx, dh, svals, dvals,
               sem, sem2):
    c = lax.axis_index("c")
    s = lax.axis_index("s")
    wid = s * NCORES + c
    ebase = wid * B1_EDGES
    rbase = s * ROWS_PER_TILE

    pltpu.sync_copy(k64_hbm, kv)
    pltpu.sync_copy(asdf_hbm.at[pl.ds(rbase * 8, ROWS_PER_TILE * 8)],
                    asd_sh.at[pl.ds(rbase * 8, ROWS_PER_TILE * 8)])
    for q in range(512 // 16):
        zv[pl.ds(q * 16, 16)] = jnp.zeros((16,), _f32)
    for j in range(ROWS_PER_TILE * H // 512):
        pltpu.sync_copy(zv, den_sh.at[pl.ds(rbase * H + j * 512, 512)])
    pltpu.sync_copy(src1d.at[pl.ds(wid * B1_EDGES, B1_EDGES)], sidx1)
    pltpu.sync_copy(dst1d.at[pl.ds(wid * B1_EDGES, B1_EDGES)], didx1)
    plsc.subcore_barrier()

    def batch(b, _):
        base = b * 128
        svs = [sidx1[pl.ds(base + g * 16, 16)] for g in range(8)]
        dvs = [didx1[pl.ds(base + g * 16, 16)] for g in range(8)]
        for h in range(H):
            kvh = kv[pl.ds(h * 16, 16)]
            for g in range(8):
                gsidx[pl.ds(g * 16, 16)] = svs[g] * 8 + h
                gdidx[pl.ds(g * 16, 16)] = dvs[g] * 8 + (h + 4)
            cps = pltpu.async_copy(asd_sh.at[gsidx], svals, sem)
            cpd = pltpu.async_copy(asd_sh.at[gdidx], dvals, sem2)
            cps.wait()
            cpd.wait()
            for g in range(8):
                e = svals[pl.ds(g * 16, 16)] + dvals[pl.ds(g * 16, 16)]
                e = jnp.where(e >= 0.0, e, 0.2 * e)
                p = jnp.exp(e - kvh)
                pbuf[pl.ds(h * B1_EDGES + base + g * 16, 16)] = p
                dh[pl.ds(g * 16, 16)] = dvs[g] * H + h
            pltpu.sync_copy(pbuf.at[pl.ds(h * B1_EDGES + base, 128)],
                            den_sh.at[dh], add=True)
        return 0

    lax.fori_loop(0, B1_BATCHES, batch, 0)

    for h in range(H):
        pltpu.sync_copy(pbuf.at[pl.ds(h * B1_EDGES, B1_EDGES)],
                        p_out.at[pl.ds(h * E_PAD + ebase, B1_EDGES)])
    plsc.subcore_barrier()
    pltpu.sync_copy(den_sh.at[pl.ds(rbase * H, ROWS_PER_TILE * H)],
                    denom2.at[c].at[pl.ds(rbase * H, ROWS_PER_TILE * H)])


def _edge_phase(src1d, dst1d, asdf, k64):
    mesh = plsc.VectorSubcoreMesh(core_axis_name="c", subcore_axis_name="s",
                                  num_cores=NCORES, num_subcores=NSUB)
    f = pl.kernel(
        _edge_body,
        out_type=[
            jax.ShapeDtypeStruct((H * E_PAD,), _f32),
            jax.ShapeDtypeStruct((NCORES, N_PAD * H), _f32),
        ],
        mesh=mesh,
        scratch_types=[
            pltpu.VMEM_SHARED((N_PAD * 8,), _f32),
            pltpu.VMEM_SHARED((N_PAD * H,), _f32),
            pltpu.VMEM((B1_EDGES,), _i32),
            pltpu.VMEM((B1_EDGES,), _i32),
            pltpu.VMEM((H * B1_EDGES,), _f32),
            pltpu.VMEM((64,), _f32),
            pltpu.VMEM((512,), _f32),
            pltpu.VMEM((128,), _i32),
            pltpu.VMEM((128,), _i32),
            pltpu.VMEM((128,), _i32),
            pltpu.VMEM((128,), _f32),
            pltpu.VMEM((128,), _f32),
            pltpu.SemaphoreType.DMA,
            pltpu.SemaphoreType.DMA,
        ],
    )
    return f(src1d, dst1d, asdf, k64)


def _msg_body(hcflat, src1d, dst1d, p_hbm, rflat_hbm, bias_hbm,
              outc,
              r_sh, acc_sh,
              abuf, exbuf, sidxb, didxb, gidx, ridx, rvals,
              rows, bbuf, biasv, sem):
    c = lax.axis_index("c")
    s = lax.axis_index("s")
    rbase = s * ROWS_PER_TILE
    tbase = s * C_EDGES

    seg = N_PAD * H // NSUB
    pltpu.sync_copy(rflat_hbm.at[pl.ds(s * seg, seg)],
                    r_sh.at[pl.ds(s * seg, seg)])
    plsc.subcore_barrier()

    for hl in range(2):
        h_abs = c * 2 + hl

        def alpha_batch(b, _):
            base = b * 128
            pltpu.sync_copy(dst1d.at[pl.ds(tbase + base, 128)], didxb)
            for g in range(8):
                dv = didxb[pl.ds(g * 16, 16)]
                ridx[pl.ds(g * 16, 16)] = dv * H + h_abs
            cpr = pltpu.async_copy(r_sh.at[ridx], rvals, sem)
            pltpu.sync_copy(p_hbm.at[pl.ds(h_abs * E_PAD + tbase + base, 128)],
                            exbuf)
            cpr.wait()
            for g in range(8):
                av = exbuf[pl.ds(g * 16, 16)] * rvals[pl.ds(g * 16, 16)]
                abuf[pl.ds(base + g * 16, 16)] = av
            return 0

        lax.fori_loop(0, C_BATCHES, alpha_batch, 0)

        for kk in range(2):
            k = hl * 2 + kk
            cg = c * (NCHUNK // NCORES) + k
            pltpu.sync_copy(bias_hbm.at[pl.ds(cg * FC, FC)], biasv)
            bvs = [biasv[pl.ds(q * 16, 16)] for q in range(FC // 16)]

            def fill(rr, _):
                for q in range(FC // 16):
                    bbuf[rr, pl.ds(q * 16, 16)] = bvs[q]
                return 0

            lax.fori_loop(0, 16, fill, 0)
            for j in range(ROWS_PER_TILE // 16):
                pltpu.sync_copy(bbuf, acc_sh.at[pl.ds(rbase + j * 16, 16)])
            plsc.subcore_barrier()

            def msg_batch(b, _):
                base = b * 128
                cps = pltpu.async_copy(src1d.at[pl.ds(tbase + base, 128)],
                                       sidxb, sem)
                pltpu.sync_copy(dst1d.at[pl.ds(tbase + base, 128)], didxb)
                cps.wait()
                for g in range(8):
                    sv = sidxb[pl.ds(g * 16, 16)]
                    gidx[pl.ds(g * 16, 16)] = sv + cg * N_PAD
                pltpu.async_copy(hcflat.at[gidx], rows, sem).wait()

                def edge(eg, _):
                    avs = abuf[pl.ds(base + eg * 16, 16)]
                    for j in range(16):
                        e = eg * 16 + j
                        av = avs[j]
                        for q in range(FC // 16):
                            hv = rows[e, pl.ds(q * 16, 16)]
                            rows[e, pl.ds(q * 16, 16)] = hv * av
                    return 0

                lax.fori_loop(0, 8, edge, 0)
                pltpu.sync_copy(rows, acc_sh.at[didxb], add=True)
                return 0

            lax.fori_loop(0, C_BATCHES, msg_batch, 0)
            plsc.subcore_barrier()
            pltpu.sync_copy(acc_sh.at[pl.ds(rbase, ROWS_PER_TILE)],
                            outc.at[cg].at[pl.ds(rbase, ROWS_PER_TILE)])
            plsc.subcore_barrier()


def _msg_phase(hcflat, src1d, dst1d, p, rflat, bias):
    mesh = plsc.VectorSubcoreMesh(core_axis_name="c", subcore_axis_name="s",
                                  num_cores=NCORES, num_subcores=NSUB)
    f = pl.kernel(
        _msg_body,
        out_type=jax.ShapeDtypeStruct((NCHUNK, N_PAD, FC), _f32),
        mesh=mesh,
        scratch_types=[
            pltpu.VMEM_SHARED((N_PAD * H,), _f32),
            pltpu.VMEM_SHARED((N_PAD, FC), _f32),
            pltpu.VMEM((C_EDGES,), _f32),
            pltpu.VMEM((128,), _f32),
            pltpu.VMEM((128,), _i32),
            pltpu.VMEM((128,), _i32),
            pltpu.VMEM((128,), _i32),
            pltpu.VMEM((128,), _i32),
            pltpu.VMEM((128,), _f32),
            pltpu.VMEM((128, FC), _f32),
            pltpu.VMEM((16, FC), _f32),
            pltpu.VMEM((FC,), _f32),
            pltpu.SemaphoreType.DMA,
        ],
    )
    return f(hcflat, src1d, dst1d, p, rflat, bias)


def kernel(x, edge_index, W, att_src, att_dst, bias):
    n = N
    wr = W.reshape(D, H, C)
    a_src = jnp.einsum("dhc,hc->dh", wr, att_src)
    a_dst = jnp.einsum("dhc,hc->dh", wr, att_dst)
    a8 = jnp.concatenate([a_src, a_dst], axis=1)
    a8p = jnp.pad(a8, ((0, 0), (0, 120)))

    x_pad = jnp.pad(x, ((0, N_PAD - n), (0, 0)))

    src = edge_index[0]
    dst = edge_index[1]
    loop = jnp.arange(n, dtype=src.dtype)
    pad_cnt = E_PAD - (src.shape[0] + n)
    pad_i = jnp.arange(pad_cnt, dtype=src.dtype)
    src_p = jnp.concatenate([src, loop, pad_i % n])
    dst_p = jnp.concatenate([dst, loop, n + (pad_i % (N_PAD - n))])

    wc = jnp.transpose(W.reshape(D, NCHUNK, FC), (1, 0, 2))
    hc = _matmul_chunks(x_pad, wc)
    a_full, amax = _logits(x_pad, a8p)
    asdf = a_full[:, :8].reshape(N_PAD * 8)
    k4 = (jnp.maximum(amax[0, :4], 0.0) + jnp.maximum(amax[0, 4:8], 0.0))
    k64 = jnp.repeat(k4, 16)

    p, denom2 = _edge_phase(src_p, dst_p, asdf, k64)
    rflat = 1.0 / (denom2[0] + denom2[1] + 1e-16)

    hcflat = hc.reshape(NCHUNK * N_PAD, FC)
    outc = _msg_phase(hcflat, src_p, dst_p, p, rflat, bias)

    out = jnp.transpose(outc[:, :n, :], (1, 0, 2)).reshape(n, HC)
    return out


# speedup vs baseline: 16.2215x; 1.2065x over previous
import functools

import jax
import jax.numpy as jnp
from jax import lax
from jax.experimental import pallas as pl
from jax.experimental.pallas import tpu as pltpu
from jax.experimental.pallas import tpu_sc as plsc

N = 10000
D = 256
H = 4
C = 256
HC = H * C
NCHUNK = 8
FC = HC // NCHUNK
N_PAD = 10240
E_PAD = 196608
BN = 512

NCORES = 2
NSUB = 16

B1_BATCHES = E_PAD // (NCORES * NSUB) // 128
B1_EDGES = B1_BATCHES * 128
C_BATCHES = E_PAD // NSUB // 128
C_EDGES = C_BATCHES * 128
ROWS_PER_TILE = N_PAD // NSUB

_i32 = jnp.int32
_f32 = jnp.float32


def _iota16():
    return lax.iota(_i32, 16)


def _mm_body(x_ref, w_ref, out_ref):
    out_ref[...] = jnp.dot(
        x_ref[...], w_ref[0], preferred_element_type=_f32
    )[None]


def _matmul_chunks(x_pad, wc):
    grid = (N_PAD // BN, NCHUNK)
    return pl.pallas_call(
        _mm_body,
        grid=grid,
        in_specs=[
            pl.BlockSpec((BN, D), lambda i, c: (i, 0)),
            pl.BlockSpec((1, D, FC), lambda i, c: (c, 0, 0)),
        ],
        out_specs=pl.BlockSpec((1, BN, FC), lambda i, c: (c, i, 0)),
        out_shape=jax.ShapeDtypeStruct((NCHUNK, N_PAD, FC), _f32),
    )(x_pad, wc)


def _logits_body(x_ref, a_ref, out_ref, mx_ref):
    i = pl.program_id(0)
    a = jnp.dot(x_ref[...], a_ref[...], preferred_element_type=_f32)
    out_ref[...] = a
    bm = jnp.max(a, axis=0, keepdims=True)

    @pl.when(i == 0)
    def _():
        mx_ref[...] = jnp.full((1, 128), -3e38, _f32)

    mx_ref[...] = jnp.maximum(mx_ref[...], bm)


def _logits(x_pad, a8p):
    return pl.pallas_call(
        _logits_body,
        grid=(N_PAD // BN,),
        in_specs=[
            pl.BlockSpec((BN, D), lambda i: (i, 0)),
            pl.BlockSpec((D, 128), lambda i: (0, 0)),
        ],
        out_specs=[
            pl.BlockSpec((BN, 128), lambda i: (i, 0)),
            pl.BlockSpec((1, 128), lambda i: (0, 0)),
        ],
        out_shape=[
            jax.ShapeDtypeStruct((N_PAD, 128), _f32),
            jax.ShapeDtypeStruct((1, 128), _f32),
        ],
    )(x_pad, a8p)


def _edge_body(src1d, dst1d, asdf_hbm, k64_hbm,
               p_out, denom2,
               asd_sh, den_sh,
               sidx1, didx1, pbuf, kv, zv, gsidx, gdidx, dh, svals, dvals,
               sem, sem2):
    c = lax.axis_index("c")
    s = lax.axis_index("s")
    wid = s * NCORES + c
    ebase = wid * B1_EDGES
    rbase = s * ROWS_PER_TILE

    pltpu.sync_copy(k64_hbm, kv)
    pltpu.sync_copy(asdf_hbm.at[pl.ds(rbase * 8, ROWS_PER_TILE * 8)],
                    asd_sh.at[pl.ds(rbase * 8, ROWS_PER_TILE * 8)])
    for q in range(512 // 16):
        zv[pl.ds(q * 16, 16)] = jnp.zeros((16,), _f32)
    for j in range(ROWS_PER_TILE * H // 512):
        pltpu.sync_copy(zv, den_sh.at[pl.ds(rbase * H + j * 512, 512)])
    pltpu.sync_copy(src1d.at[pl.ds(wid * B1_EDGES, B1_EDGES)], sidx1)
    pltpu.sync_copy(dst1d.at[pl.ds(wid * B1_EDGES, B1_EDGES)], didx1)
    plsc.subcore_barrier()

    def batch(b, _):
        base = b * 128
        svs = [sidx1[pl.ds(base + g * 16, 16)] for g in range(8)]
        dvs = [didx1[pl.ds(base + g * 16, 16)] for g in range(8)]
        for h in range(H):
            kvh = kv[pl.ds(h * 16, 16)]
            for g in range(8):
                gsidx[pl.ds(g * 16, 16)] = svs[g] * 8 + h
                gdidx[pl.ds(g * 16, 16)] = dvs[g] * 8 + (h + 4)
            cps = pltpu.async_copy(asd_sh.at[gsidx], svals, sem)
            cpd = pltpu.async_copy(asd_sh.at[gdidx], dvals, sem2)
            cps.wait()
            cpd.wait()
            for g in range(8):
                e = svals[pl.ds(g * 16, 16)] + dvals[pl.ds(g * 16, 16)]
                e = jnp.where(e >= 0.0, e, 0.2 * e)
                p = jnp.exp(e - kvh)
                pbuf[pl.ds(h * B1_EDGES + base + g * 16, 16)] = p
                dh[pl.ds(g * 16, 16)] = dvs[g] * H + h
            pltpu.sync_copy(pbuf.at[pl.ds(h * B1_EDGES + base, 128)],
                            den_sh.at[dh], add=True)
        return 0

    lax.fori_loop(0, B1_BATCHES, batch, 0)

    for h in range(H):
        pltpu.sync_copy(pbuf.at[pl.ds(h * B1_EDGES, B1_EDGES)],
                        p_out.at[pl.ds(h * E_PAD + ebase, B1_EDGES)])
    plsc.subcore_barrier()
    pltpu.sync_copy(den_sh.at[pl.ds(rbase * H, ROWS_PER_TILE * H)],
                    denom2.at[c].at[pl.ds(rbase * H, ROWS_PER_TILE * H)])


def _edge_phase(src1d, dst1d, asdf, k64):
    mesh = plsc.VectorSubcoreMesh(core_axis_name="c", subcore_axis_name="s",
                                  num_cores=NCORES, num_subcores=NSUB)
    f = pl.kernel(
        _edge_body,
        out_type=[
            jax.ShapeDtypeStruct((H * E_PAD,), _f32),
            jax.ShapeDtypeStruct((NCORES, N_PAD * H), _f32),
        ],
        mesh=mesh,
        scratch_types=[
            pltpu.VMEM_SHARED((N_PAD * 8,), _f32),
            pltpu.VMEM_SHARED((N_PAD * H,), _f32),
            pltpu.VMEM((B1_EDGES,), _i32),
            pltpu.VMEM((B1_EDGES,), _i32),
            pltpu.VMEM((H * B1_EDGES,), _f32),
            pltpu.VMEM((64,), _f32),
            pltpu.VMEM((512,), _f32),
            pltpu.VMEM((128,), _i32),
            pltpu.VMEM((128,), _i32),
            pltpu.VMEM((128,), _i32),
            pltpu.VMEM((128,), _f32),
            pltpu.VMEM((128,), _f32),
            pltpu.SemaphoreType.DMA,
            pltpu.SemaphoreType.DMA,
        ],
    )
    return f(src1d, dst1d, asdf, k64)


def _msg_body(hcflat, src1d, dst1d, p_hbm, rflat_hbm, bias_hbm,
              outc,
              r_sh, acc_sh,
              abuf, exbuf, ridx, rvals,
              sidx0, sidx1b, didx0, didx1b, gidx0, gidx1b,
              rows0, rows1, biasv, sem, gsem0, gsem1, ssem0, ssem1):
    c = lax.axis_index("c")
    s = lax.axis_index("s")
    rbase = s * ROWS_PER_TILE
    tbase = s * C_EDGES

    seg = N_PAD * H // NSUB
    pltpu.sync_copy(rflat_hbm.at[pl.ds(s * seg, seg)],
                    r_sh.at[pl.ds(s * seg, seg)])
    plsc.subcore_barrier()

    for hl in range(2):
        h_abs = c * 2 + hl

        def alpha_batch(b, _):
            base = b * 128
            pltpu.sync_copy(dst1d.at[pl.ds(tbase + base, 128)], didx0)
            for g in range(8):
                dv = didx0[pl.ds(g * 16, 16)]
                ridx[pl.ds(g * 16, 16)] = dv * H + h_abs
            cpr = pltpu.async_copy(r_sh.at[ridx], rvals, sem)
            pltpu.sync_copy(p_hbm.at[pl.ds(h_abs * E_PAD + tbase + base, 128)],
                            exbuf)
            cpr.wait()
            for g in range(8):
                av = exbuf[pl.ds(g * 16, 16)] * rvals[pl.ds(g * 16, 16)]
                abuf[pl.ds(base + g * 16, 16)] = av
            return 0

        lax.fori_loop(0, C_BATCHES, alpha_batch, 0)

        for kk in range(2):
            k = hl * 2 + kk
            cg = c * (NCHUNK // NCORES) + k
            pltpu.sync_copy(bias_hbm.at[pl.ds(cg * FC, FC)], biasv)
            bvs = [biasv[pl.ds(q * 16, 16)] for q in range(FC // 16)]

            def fill(rr, _):
                for q in range(FC // 16):
                    rows0[rr, pl.ds(q * 16, 16)] = bvs[q]
                return 0

            lax.fori_loop(0, 128, fill, 0)
            for j in range(ROWS_PER_TILE // 128):
                pltpu.sync_copy(rows0, acc_sh.at[pl.ds(rbase + j * 128, 128)])
            plsc.subcore_barrier()

            rowsb = (rows0, rows1)
            sidxs = (sidx0, sidx1b)
            didxs = (didx0, didx1b)
            gidxs = (gidx0, gidx1b)
            gsems = (gsem0, gsem1)

            def stage_and_gather(b, par):
                base = b * 128
                pltpu.sync_copy(dst1d.at[pl.ds(tbase + base, 128)],
                                didxs[par])
                pltpu.sync_copy(src1d.at[pl.ds(tbase + base, 128)],
                                sidxs[par])
                for g in range(8):
                    sv = sidxs[par][pl.ds(g * 16, 16)]
                    gidxs[par][pl.ds(g * 16, 16)] = sv + cg * N_PAD
                return pltpu.async_copy(hcflat.at[gidxs[par]], rowsb[par],
                                        gsems[par])

            def compute_scatter(b, par):
                def edge(eg, _):
                    avs = abuf[pl.ds(b * 128 + eg * 16, 16)]
                    for j in range(16):
                        e = eg * 16 + j
                        av = avs[j]
                        for q in range(FC // 16):
                            hv = rowsb[par][e, pl.ds(q * 16, 16)]
                            rowsb[par][e, pl.ds(q * 16, 16)] = hv * av
                    return 0

                lax.fori_loop(0, 8, edge, 0)
                pltpu.sync_copy(rowsb[par], acc_sh.at[didxs[par]], add=True)

            stage_and_gather(0, 0).wait()

            def step(st, _):
                b0 = st * 2
                d1 = stage_and_gather(b0 + 1, 1)
                compute_scatter(b0, 0)
                d1.wait()
                d0 = stage_and_gather(b0 + 2, 0)
                compute_scatter(b0 + 1, 1)
                d0.wait()
                return 0

            lax.fori_loop(0, C_BATCHES // 2 - 1, step, 0)
            d1 = stage_and_gather(C_BATCHES - 1, 1)
            compute_scatter(C_BATCHES - 2, 0)
            d1.wait()
            compute_scatter(C_BATCHES - 1, 1)
            plsc.subcore_barrier()
            pltpu.sync_copy(acc_sh.at[pl.ds(rbase, ROWS_PER_TILE)],
                            outc.at[cg].at[pl.ds(rbase, ROWS_PER_TILE)])
            plsc.subcore_barrier()


def _msg_phase(hcflat, src1d, dst1d, p, rflat, bias):
    mesh = plsc.VectorSubcoreMesh(core_axis_name="c", subcore_axis_name="s",
                                  num_cores=NCORES, num_subcores=NSUB)
    f = pl.kernel(
        _msg_body,
        out_type=jax.ShapeDtypeStruct((NCHUNK, N_PAD, FC), _f32),
        mesh=mesh,
        scratch_types=[
            pltpu.VMEM_SHARED((N_PAD * H,), _f32),
            pltpu.VMEM_SHARED((N_PAD, FC), _f32),
            pltpu.VMEM((C_EDGES,), _f32),
            pltpu.VMEM((128,), _f32),
            pltpu.VMEM((128,), _i32),
            pltpu.VMEM((128,), _f32),
            pltpu.VMEM((128,), _i32),
            pltpu.VMEM((128,), _i32),
            pltpu.VMEM((128,), _i32),
            pltpu.VMEM((128,), _i32),
            pltpu.VMEM((128,), _i32),
            pltpu.VMEM((128,), _i32),
            pltpu.VMEM((128, FC), _f32),
            pltpu.VMEM((128, FC), _f32),
            pltpu.VMEM((FC,), _f32),
            pltpu.SemaphoreType.DMA,
            pltpu.SemaphoreType.DMA,
            pltpu.SemaphoreType.DMA,
            pltpu.SemaphoreType.DMA,
            pltpu.SemaphoreType.DMA,
        ],
    )
    return f(hcflat, src1d, dst1d, p, rflat, bias)


def kernel(x, edge_index, W, att_src, att_dst, bias):
    n = N
    wr = W.reshape(D, H, C)
    a_src = jnp.einsum("dhc,hc->dh", wr, att_src)
    a_dst = jnp.einsum("dhc,hc->dh", wr, att_dst)
    a8 = jnp.concatenate([a_src, a_dst], axis=1)
    a8p = jnp.pad(a8, ((0, 0), (0, 120)))

    x_pad = jnp.pad(x, ((0, N_PAD - n), (0, 0)))

    src = edge_index[0]
    dst = edge_index[1]
    loop = jnp.arange(n, dtype=src.dtype)
    pad_cnt = E_PAD - (src.shape[0] + n)
    pad_i = jnp.arange(pad_cnt, dtype=src.dtype)
    src_p = jnp.concatenate([src, loop, pad_i % n])
    dst_p = jnp.concatenate([dst, loop, n + (pad_i % (N_PAD - n))])

    wc = jnp.transpose(W.reshape(D, NCHUNK, FC), (1, 0, 2))
    hc = _matmul_chunks(x_pad, wc)
    a_full, amax = _logits(x_pad, a8p)
    asdf = a_full[:, :8].reshape(N_PAD * 8)
    k4 = (jnp.maximum(amax[0, :4], 0.0) + jnp.maximum(amax[0, 4:8], 0.0))
    k64 = jnp.repeat(k4, 16)

    p, denom2 = _edge_phase(src_p, dst_p, asdf, k64)
    rflat = 1.0 / (denom2[0] + denom2[1] + 1e-16)

    hcflat = hc.reshape(NCHUNK * N_PAD, FC)
    outc = _msg_phase(hcflat, src_p, dst_p, p, rflat, bias)

    out = jnp.transpose(outc[:, :n, :], (1, 0, 2)).reshape(n, HC)
    return out


# speedup vs baseline: 17.9868x; 1.1088x over previous
import functools

import jax
import jax.numpy as jnp
from jax import lax
from jax.experimental import pallas as pl
from jax.experimental.pallas import tpu as pltpu
from jax.experimental.pallas import tpu_sc as plsc

N = 10000
D = 256
H = 4
C = 256
HC = H * C
NCHUNK = 8
FC = HC // NCHUNK
N_PAD = 10240
E_PAD = 172032
BN = 512

NCORES = 2
NSUB = 16

B1_BATCHES = E_PAD // (NCORES * NSUB) // 128
B1_EDGES = B1_BATCHES * 128
C_BATCHES = E_PAD // NSUB // 128
C_EDGES = C_BATCHES * 128
ROWS_PER_TILE = N_PAD // NSUB

_i32 = jnp.int32
_f32 = jnp.float32


def _iota16():
    return lax.iota(_i32, 16)


def _mm_body(x_ref, w_ref, out_ref):
    out_ref[...] = jnp.dot(
        x_ref[...], w_ref[0], preferred_element_type=_f32
    )[None]


def _matmul_chunks(x_pad, wc):
    grid = (N_PAD // BN, NCHUNK)
    return pl.pallas_call(
        _mm_body,
        grid=grid,
        in_specs=[
            pl.BlockSpec((BN, D), lambda i, c: (i, 0)),
            pl.BlockSpec((1, D, FC), lambda i, c: (c, 0, 0)),
        ],
        out_specs=pl.BlockSpec((1, BN, FC), lambda i, c: (c, i, 0)),
        out_shape=jax.ShapeDtypeStruct((NCHUNK, N_PAD, FC), _f32),
    )(x_pad, wc)


def _logits_body(x_ref, a_ref, out_ref, mx_ref):
    i = pl.program_id(0)
    a = jnp.dot(x_ref[...], a_ref[...], preferred_element_type=_f32)
    out_ref[...] = a
    bm = jnp.max(a, axis=0, keepdims=True)

    @pl.when(i == 0)
    def _():
        mx_ref[...] = jnp.full((1, 128), -3e38, _f32)

    mx_ref[...] = jnp.maximum(mx_ref[...], bm)


def _logits(x_pad, a8p):
    return pl.pallas_call(
        _logits_body,
        grid=(N_PAD // BN,),
        in_specs=[
            pl.BlockSpec((BN, D), lambda i: (i, 0)),
            pl.BlockSpec((D, 128), lambda i: (0, 0)),
        ],
        out_specs=[
            pl.BlockSpec((BN, 128), lambda i: (i, 0)),
            pl.BlockSpec((1, 128), lambda i: (0, 0)),
        ],
        out_shape=[
            jax.ShapeDtypeStruct((N_PAD, 128), _f32),
            jax.ShapeDtypeStruct((1, 128), _f32),
        ],
    )(x_pad, a8p)


def _edge_body(src1d, dst1d, asdf_hbm, k64_hbm,
               p_out, denom2,
               asd_sh, den_sh,
               sidx1, didx1, pbuf, kv, zv, gsidx, gdidx, dh, svals, dvals,
               sem, sem2):
    c = lax.axis_index("c")
    s = lax.axis_index("s")
    wid = s * NCORES + c
    ebase = wid * B1_EDGES
    rbase = s * ROWS_PER_TILE

    pltpu.sync_copy(k64_hbm, kv)
    pltpu.sync_copy(asdf_hbm.at[pl.ds(rbase * 8, ROWS_PER_TILE * 8)],
                    asd_sh.at[pl.ds(rbase * 8, ROWS_PER_TILE * 8)])
    for q in range(512 // 16):
        zv[pl.ds(q * 16, 16)] = jnp.zeros((16,), _f32)
    for j in range(ROWS_PER_TILE * H // 512):
        pltpu.sync_copy(zv, den_sh.at[pl.ds(rbase * H + j * 512, 512)])
    pltpu.sync_copy(src1d.at[pl.ds(wid * B1_EDGES, B1_EDGES)], sidx1)
    pltpu.sync_copy(dst1d.at[pl.ds(wid * B1_EDGES, B1_EDGES)], didx1)
    plsc.subcore_barrier()

    def batch(b, _):
        base = b * 128
        svs = [sidx1[pl.ds(base + g * 16, 16)] for g in range(8)]
        dvs = [didx1[pl.ds(base + g * 16, 16)] for g in range(8)]
        for h in range(H):
            kvh = kv[pl.ds(h * 16, 16)]
            for g in range(8):
                gsidx[pl.ds(g * 16, 16)] = svs[g] * 8 + h
                gdidx[pl.ds(g * 16, 16)] = dvs[g] * 8 + (h + 4)
            cps = pltpu.async_copy(asd_sh.at[gsidx], svals, sem)
            cpd = pltpu.async_copy(asd_sh.at[gdidx], dvals, sem2)
            cps.wait()
            cpd.wait()
            for g in range(8):
                e = svals[pl.ds(g * 16, 16)] + dvals[pl.ds(g * 16, 16)]
                e = jnp.where(e >= 0.0, e, 0.2 * e)
                p = jnp.exp(e - kvh)
                pbuf[pl.ds(h * B1_EDGES + base + g * 16, 16)] = p
                dh[pl.ds(g * 16, 16)] = dvs[g] * H + h
            pltpu.sync_copy(pbuf.at[pl.ds(h * B1_EDGES + base, 128)],
                            den_sh.at[dh], add=True)
        return 0

    lax.fori_loop(0, B1_BATCHES, batch, 0)

    for h in range(H):
        pltpu.sync_copy(pbuf.at[pl.ds(h * B1_EDGES, B1_EDGES)],
                        p_out.at[pl.ds(h * E_PAD + ebase, B1_EDGES)])
    plsc.subcore_barrier()
    pltpu.sync_copy(den_sh.at[pl.ds(rbase * H, ROWS_PER_TILE * H)],
                    denom2.at[c].at[pl.ds(rbase * H, ROWS_PER_TILE * H)])


def _edge_phase(src1d, dst1d, asdf, k64):
    mesh = plsc.VectorSubcoreMesh(core_axis_name="c", subcore_axis_name="s",
                                  num_cores=NCORES, num_subcores=NSUB)
    f = pl.kernel(
        _edge_body,
        out_type=[
            jax.ShapeDtypeStruct((H * E_PAD,), _f32),
            jax.ShapeDtypeStruct((NCORES, N_PAD * H), _f32),
        ],
        mesh=mesh,
        scratch_types=[
            pltpu.VMEM_SHARED((N_PAD * 8,), _f32),
            pltpu.VMEM_SHARED((N_PAD * H,), _f32),
            pltpu.VMEM((B1_EDGES,), _i32),
            pltpu.VMEM((B1_EDGES,), _i32),
            pltpu.VMEM((H * B1_EDGES,), _f32),
            pltpu.VMEM((64,), _f32),
            pltpu.VMEM((512,), _f32),
            pltpu.VMEM((128,), _i32),
            pltpu.VMEM((128,), _i32),
            pltpu.VMEM((128,), _i32),
            pltpu.VMEM((128,), _f32),
            pltpu.VMEM((128,), _f32),
            pltpu.SemaphoreType.DMA,
            pltpu.SemaphoreType.DMA,
        ],
    )
    return f(src1d, dst1d, asdf, k64)


def _msg_body(hcflat, src1d, dst1d, p_hbm, rflat_hbm, bias_hbm,
              outc,
              r_sh, acc_sh,
              abuf, exbuf, ridx, rvals,
              sidx0, sidx1b, didx0, didx1b, gidx0, gidx1b,
              rows0, rows1, biasv, sem, gsem0, gsem1, ssem0, ssem1):
    c = lax.axis_index("c")
    s = lax.axis_index("s")
    rbase = s * ROWS_PER_TILE
    tbase = s * C_EDGES

    seg = N_PAD * H // NSUB
    pltpu.sync_copy(rflat_hbm.at[pl.ds(s * seg, seg)],
                    r_sh.at[pl.ds(s * seg, seg)])
    plsc.subcore_barrier()

    for hl in range(2):
        h_abs = c * 2 + hl

        def alpha_batch(b, _):
            base = b * 128
            pltpu.sync_copy(dst1d.at[pl.ds(tbase + base, 128)], didx0)
            for g in range(8):
                dv = didx0[pl.ds(g * 16, 16)]
                ridx[pl.ds(g * 16, 16)] = dv * H + h_abs
            cpr = pltpu.async_copy(r_sh.at[ridx], rvals, sem)
            pltpu.sync_copy(p_hbm.at[pl.ds(h_abs * E_PAD + tbase + base, 128)],
                            exbuf)
            cpr.wait()
            for g in range(8):
                av = exbuf[pl.ds(g * 16, 16)] * rvals[pl.ds(g * 16, 16)]
                abuf[pl.ds(base + g * 16, 16)] = av
            return 0

        lax.fori_loop(0, C_BATCHES, alpha_batch, 0)

        for kk in range(2):
            k = hl * 2 + kk
            cg = c * (NCHUNK // NCORES) + k
            pltpu.sync_copy(bias_hbm.at[pl.ds(cg * FC, FC)], biasv)
            bvs = [biasv[pl.ds(q * 16, 16)] for q in range(FC // 16)]

            def fill(rr, _):
                for q in range(FC // 16):
                    rows0[rr, pl.ds(q * 16, 16)] = bvs[q]
                return 0

            lax.fori_loop(0, 128, fill, 0)
            for j in range(ROWS_PER_TILE // 128):
                pltpu.sync_copy(rows0, acc_sh.at[pl.ds(rbase + j * 128, 128)])
            plsc.subcore_barrier()

            rowsb = (rows0, rows1)
            sidxs = (sidx0, sidx1b)
            didxs = (didx0, didx1b)
            gidxs = (gidx0, gidx1b)
            gsems = (gsem0, gsem1)

            def stage_and_gather(b, par):
                base = b * 128
                pltpu.sync_copy(dst1d.at[pl.ds(tbase + base, 128)],
                                didxs[par])
                pltpu.sync_copy(src1d.at[pl.ds(tbase + base, 128)],
                                sidxs[par])
                for g in range(8):
                    sv = sidxs[par][pl.ds(g * 16, 16)]
                    gidxs[par][pl.ds(g * 16, 16)] = sv + cg * N_PAD
                return pltpu.async_copy(hcflat.at[gidxs[par]], rowsb[par],
                                        gsems[par])

            def compute_scatter(b, par):
                def edge(eg, _):
                    avs = abuf[pl.ds(b * 128 + eg * 16, 16)]
                    for j in range(16):
                        e = eg * 16 + j
                        av = avs[j]
                        for q in range(FC // 16):
                            hv = rowsb[par][e, pl.ds(q * 16, 16)]
                            rowsb[par][e, pl.ds(q * 16, 16)] = hv * av
                    return 0

                lax.fori_loop(0, 8, edge, 0)
                pltpu.sync_copy(rowsb[par], acc_sh.at[didxs[par]], add=True)

            stage_and_gather(0, 0).wait()

            def step(st, _):
                b0 = st * 2
                d1 = stage_and_gather(b0 + 1, 1)
                compute_scatter(b0, 0)
                d1.wait()
                d0 = stage_and_gather(b0 + 2, 0)
                compute_scatter(b0 + 1, 1)
                d0.wait()
                return 0

            lax.fori_loop(0, C_BATCHES // 2 - 1, step, 0)
            d1 = stage_and_gather(C_BATCHES - 1, 1)
            compute_scatter(C_BATCHES - 2, 0)
            d1.wait()
            compute_scatter(C_BATCHES - 1, 1)
            plsc.subcore_barrier()
            pltpu.sync_copy(acc_sh.at[pl.ds(rbase, ROWS_PER_TILE)],
                            outc.at[cg].at[pl.ds(rbase, ROWS_PER_TILE)])
            plsc.subcore_barrier()


def _msg_phase(hcflat, src1d, dst1d, p, rflat, bias):
    mesh = plsc.VectorSubcoreMesh(core_axis_name="c", subcore_axis_name="s",
                                  num_cores=NCORES, num_subcores=NSUB)
    f = pl.kernel(
        _msg_body,
        out_type=jax.ShapeDtypeStruct((NCHUNK, N_PAD, FC), _f32),
        mesh=mesh,
        scratch_types=[
            pltpu.VMEM_SHARED((N_PAD * H,), _f32),
            pltpu.VMEM_SHARED((N_PAD, FC), _f32),
            pltpu.VMEM((C_EDGES,), _f32),
            pltpu.VMEM((128,), _f32),
            pltpu.VMEM((128,), _i32),
            pltpu.VMEM((128,), _f32),
            pltpu.VMEM((128,), _i32),
            pltpu.VMEM((128,), _i32),
            pltpu.VMEM((128,), _i32),
            pltpu.VMEM((128,), _i32),
            pltpu.VMEM((128,), _i32),
            pltpu.VMEM((128,), _i32),
            pltpu.VMEM((128, FC), _f32),
            pltpu.VMEM((128, FC), _f32),
            pltpu.VMEM((FC,), _f32),
            pltpu.SemaphoreType.DMA,
            pltpu.SemaphoreType.DMA,
            pltpu.SemaphoreType.DMA,
            pltpu.SemaphoreType.DMA,
            pltpu.SemaphoreType.DMA,
        ],
    )
    return f(hcflat, src1d, dst1d, p, rflat, bias)


def kernel(x, edge_index, W, att_src, att_dst, bias):
    n = N
    wr = W.reshape(D, H, C)
    a_src = jnp.einsum("dhc,hc->dh", wr, att_src)
    a_dst = jnp.einsum("dhc,hc->dh", wr, att_dst)
    a8 = jnp.concatenate([a_src, a_dst], axis=1)
    a8p = jnp.pad(a8, ((0, 0), (0, 120)))

    x_pad = jnp.pad(x, ((0, N_PAD - n), (0, 0)))

    src = edge_index[0]
    dst = edge_index[1]
    loop = jnp.arange(n, dtype=src.dtype)
    pad_cnt = E_PAD - (src.shape[0] + n)
    pad_i = jnp.arange(pad_cnt, dtype=src.dtype)
    src_p = jnp.concatenate([src, loop, pad_i % n])
    dst_p = jnp.concatenate([dst, loop, n + (pad_i % (N_PAD - n))])

    wc = jnp.transpose(W.reshape(D, NCHUNK, FC), (1, 0, 2))
    hc = _matmul_chunks(x_pad, wc)
    a_full, amax = _logits(x_pad, a8p)
    asdf = a_full[:, :8].reshape(N_PAD * 8)
    k4 = (jnp.maximum(amax[0, :4], 0.0) + jnp.maximum(amax[0, 4:8], 0.0))
    k64 = jnp.repeat(k4, 16)

    p, denom2 = _edge_phase(src_p, dst_p, asdf, k64)
    rflat = 1.0 / (denom2[0] + denom2[1] + 1e-16)

    hcflat = hc.reshape(NCHUNK * N_PAD, FC)
    outc = _msg_phase(hcflat, src_p, dst_p, p, rflat, bias)

    out = jnp.transpose(outc[:, :n, :], (1, 0, 2)).reshape(n, HC)
    return out


# speedup vs baseline: 20.7449x; 1.1533x over previous
import functools

import jax
import jax.numpy as jnp
from jax import lax
from jax.experimental import pallas as pl
from jax.experimental.pallas import tpu as pltpu
from jax.experimental.pallas import tpu_sc as plsc

N = 10000
D = 256
H = 4
C = 256
HC = H * C
NCHUNK = 8
FC = HC // NCHUNK
N_PAD = 10240
E_PAD = 172032
BN = 512

NCORES = 2
NSUB = 16

B1_BATCHES = E_PAD // (NCORES * NSUB) // 128
B1_EDGES = B1_BATCHES * 128
C_BATCHES = E_PAD // NSUB // 128
C_EDGES = C_BATCHES * 128
ROWS_PER_TILE = N_PAD // NSUB

_i32 = jnp.int32
_f32 = jnp.float32


def _iota16():
    return lax.iota(_i32, 16)


def _mm_body(x_ref, w_ref, out_ref):
    out_ref[...] = jnp.dot(
        x_ref[...], w_ref[0], preferred_element_type=_f32
    )[None]


def _matmul_chunks(x_pad, wc):
    grid = (N_PAD // BN, NCHUNK)
    return pl.pallas_call(
        _mm_body,
        grid=grid,
        in_specs=[
            pl.BlockSpec((BN, D), lambda i, c: (i, 0)),
            pl.BlockSpec((1, D, FC), lambda i, c: (c, 0, 0)),
        ],
        out_specs=pl.BlockSpec((1, BN, FC), lambda i, c: (c, i, 0)),
        out_shape=jax.ShapeDtypeStruct((NCHUNK, N_PAD, FC), _f32),
    )(x_pad, wc)


def _logits_body(x_ref, a_ref, out_ref, mx_ref):
    i = pl.program_id(0)
    a = jnp.dot(x_ref[...], a_ref[...], preferred_element_type=_f32)
    out_ref[...] = a
    bm = jnp.max(a, axis=0, keepdims=True)

    @pl.when(i == 0)
    def _():
        mx_ref[...] = jnp.full((1, 128), -3e38, _f32)

    mx_ref[...] = jnp.maximum(mx_ref[...], bm)


def _logits(x_pad, a8p):
    return pl.pallas_call(
        _logits_body,
        grid=(N_PAD // BN,),
        in_specs=[
            pl.BlockSpec((BN, D), lambda i: (i, 0)),
            pl.BlockSpec((D, 128), lambda i: (0, 0)),
        ],
        out_specs=[
            pl.BlockSpec((BN, 128), lambda i: (i, 0)),
            pl.BlockSpec((1, 128), lambda i: (0, 0)),
        ],
        out_shape=[
            jax.ShapeDtypeStruct((N_PAD, 128), _f32),
            jax.ShapeDtypeStruct((1, 128), _f32),
        ],
    )(x_pad, a8p)


def _edge_body(src1d, dst1d, asdf_hbm, k64_hbm,
               p_out, denom2,
               asd_sh, den_sh,
               sidx1, didx1, pbuf, kv, zv, gsidx, gdidx, dh, svals, dvals,
               sem, sem2):
    c = lax.axis_index("c")
    s = lax.axis_index("s")
    wid = s * NCORES + c
    ebase = wid * B1_EDGES
    rbase = s * ROWS_PER_TILE

    pltpu.sync_copy(k64_hbm, kv)
    pltpu.sync_copy(asdf_hbm.at[pl.ds(rbase * 8, ROWS_PER_TILE * 8)],
                    asd_sh.at[pl.ds(rbase * 8, ROWS_PER_TILE * 8)])
    for q in range(512 // 16):
        zv[pl.ds(q * 16, 16)] = jnp.zeros((16,), _f32)
    for j in range(ROWS_PER_TILE * H // 512):
        pltpu.sync_copy(zv, den_sh.at[pl.ds(rbase * H + j * 512, 512)])
    pltpu.sync_copy(src1d.at[pl.ds(wid * B1_EDGES, B1_EDGES)], sidx1)
    pltpu.sync_copy(dst1d.at[pl.ds(wid * B1_EDGES, B1_EDGES)], didx1)
    plsc.subcore_barrier()

    def batch(b, _):
        base = b * 128
        svs = [sidx1[pl.ds(base + g * 16, 16)] for g in range(8)]
        dvs = [didx1[pl.ds(base + g * 16, 16)] for g in range(8)]
        for h in range(H):
            kvh = kv[pl.ds(h * 16, 16)]
            for g in range(8):
                gsidx[pl.ds(g * 16, 16)] = svs[g] * 8 + h
                gdidx[pl.ds(g * 16, 16)] = dvs[g] * 8 + (h + 4)
            cps = pltpu.async_copy(asd_sh.at[gsidx], svals, sem)
            cpd = pltpu.async_copy(asd_sh.at[gdidx], dvals, sem2)
            cps.wait()
            cpd.wait()
            for g in range(8):
                e = svals[pl.ds(g * 16, 16)] + dvals[pl.ds(g * 16, 16)]
                e = jnp.where(e >= 0.0, e, 0.2 * e)
                p = jnp.exp(e - kvh)
                pbuf[pl.ds(h * B1_EDGES + base + g * 16, 16)] = p
                dh[pl.ds(g * 16, 16)] = dvs[g] * H + h
            pltpu.sync_copy(pbuf.at[pl.ds(h * B1_EDGES + base, 128)],
                            den_sh.at[dh], add=True)
        return 0

    lax.fori_loop(0, B1_BATCHES, batch, 0)

    for h in range(H):
        pltpu.sync_copy(pbuf.at[pl.ds(h * B1_EDGES, B1_EDGES)],
                        p_out.at[pl.ds(h * E_PAD + ebase, B1_EDGES)])
    plsc.subcore_barrier()
    pltpu.sync_copy(den_sh.at[pl.ds(rbase * H, ROWS_PER_TILE * H)],
                    denom2.at[c].at[pl.ds(rbase * H, ROWS_PER_TILE * H)])


def _edge_phase(src1d, dst1d, asdf, k64):
    mesh = plsc.VectorSubcoreMesh(core_axis_name="c", subcore_axis_name="s",
                                  num_cores=NCORES, num_subcores=NSUB)
    f = pl.kernel(
        _edge_body,
        out_type=[
            jax.ShapeDtypeStruct((H * E_PAD,), _f32),
            jax.ShapeDtypeStruct((NCORES, N_PAD * H), _f32),
        ],
        mesh=mesh,
        scratch_types=[
            pltpu.VMEM_SHARED((N_PAD * 8,), _f32),
            pltpu.VMEM_SHARED((N_PAD * H,), _f32),
            pltpu.VMEM((B1_EDGES,), _i32),
            pltpu.VMEM((B1_EDGES,), _i32),
            pltpu.VMEM((H * B1_EDGES,), _f32),
            pltpu.VMEM((64,), _f32),
            pltpu.VMEM((512,), _f32),
            pltpu.VMEM((128,), _i32),
            pltpu.VMEM((128,), _i32),
            pltpu.VMEM((128,), _i32),
            pltpu.VMEM((128,), _f32),
            pltpu.VMEM((128,), _f32),
            pltpu.SemaphoreType.DMA,
            pltpu.SemaphoreType.DMA,
        ],
    )
    return f(src1d, dst1d, asdf, k64)


def _msg_body(hcflat, src1d, dst1d, p_hbm, rflat_hbm, bias_hbm,
              outc,
              r_sh, acc_sh,
              abuf, exbuf, ridx, rvals,
              sidx0, sidx1b, didx0, didx1b, gidx0, gidx1b,
              rows0, rows1, biasv, sem, gsem0, gsem1, isem, isem2):
    c = lax.axis_index("c")
    s = lax.axis_index("s")
    rbase = s * ROWS_PER_TILE
    tbase = s * C_EDGES

    seg = N_PAD * H // NSUB
    pltpu.sync_copy(rflat_hbm.at[pl.ds(s * seg, seg)],
                    r_sh.at[pl.ds(s * seg, seg)])
    plsc.subcore_barrier()

    for hl in range(2):
        h_abs = c * 2 + hl

        def alpha_batch(b, _):
            base = b * 128
            pltpu.sync_copy(dst1d.at[pl.ds(tbase + base, 128)], didx0)
            for g in range(8):
                dv = didx0[pl.ds(g * 16, 16)]
                ridx[pl.ds(g * 16, 16)] = dv * H + h_abs
            cpr = pltpu.async_copy(r_sh.at[ridx], rvals, sem)
            pltpu.sync_copy(p_hbm.at[pl.ds(h_abs * E_PAD + tbase + base, 128)],
                            exbuf)
            cpr.wait()
            for g in range(8):
                av = exbuf[pl.ds(g * 16, 16)] * rvals[pl.ds(g * 16, 16)]
                abuf[pl.ds(base + g * 16, 16)] = av
            return 0

        lax.fori_loop(0, C_BATCHES, alpha_batch, 0)

        for kk in range(2):
            k = hl * 2 + kk
            cg = c * (NCHUNK // NCORES) + k
            pltpu.sync_copy(bias_hbm.at[pl.ds(cg * FC, FC)], biasv)
            bvs = [biasv[pl.ds(q * 16, 16)] for q in range(FC // 16)]

            def fill(rr, _):
                for q in range(FC // 16):
                    rows0[rr, pl.ds(q * 16, 16)] = bvs[q]
                return 0

            lax.fori_loop(0, 128, fill, 0)
            for j in range(ROWS_PER_TILE // 128):
                pltpu.sync_copy(rows0, acc_sh.at[pl.ds(rbase + j * 128, 128)])
            plsc.subcore_barrier()

            rowsb = (rows0, rows1)
            sidxs = (sidx0, sidx1b)
            didxs = (didx0, didx1b)
            gidxs = (gidx0, gidx1b)
            gsems = (gsem0, gsem1)

            def stage_and_gather(b, par):
                base = b * 128
                ci = pltpu.async_copy(dst1d.at[pl.ds(tbase + base, 128)],
                                      didxs[par], isem)
                cj = pltpu.async_copy(src1d.at[pl.ds(tbase + base, 128)],
                                      sidxs[par], isem2)
                cj.wait()
                ci.wait()
                for g in range(8):
                    sv = sidxs[par][pl.ds(g * 16, 16)]
                    gidxs[par][pl.ds(g * 16, 16)] = sv + cg * N_PAD
                return pltpu.async_copy(hcflat.at[gidxs[par]], rowsb[par],
                                        gsems[par])

            def compute_scatter(b, par):
                def edge(eg, _):
                    avs = abuf[pl.ds(b * 128 + eg * 16, 16)]
                    for j in range(16):
                        e = eg * 16 + j
                        av = avs[j]
                        for q in range(FC // 16):
                            hv = rowsb[par][e, pl.ds(q * 16, 16)]
                            rowsb[par][e, pl.ds(q * 16, 16)] = hv * av
                    return 0

                lax.fori_loop(0, 8, edge, 0)
                pltpu.sync_copy(rowsb[par], acc_sh.at[didxs[par]], add=True)

            stage_and_gather(0, 0).wait()

            def step(st, _):
                b0 = st * 2
                d1 = stage_and_gather(b0 + 1, 1)
                compute_scatter(b0, 0)
                d1.wait()
                d0 = stage_and_gather(b0 + 2, 0)
                compute_scatter(b0 + 1, 1)
                d0.wait()
                return 0

            lax.fori_loop(0, C_BATCHES // 2 - 1, step, 0)
            d1 = stage_and_gather(C_BATCHES - 1, 1)
            compute_scatter(C_BATCHES - 2, 0)
            d1.wait()
            compute_scatter(C_BATCHES - 1, 1)
            plsc.subcore_barrier()
            pltpu.sync_copy(acc_sh.at[pl.ds(rbase, ROWS_PER_TILE)],
                            outc.at[pl.ds(rbase, ROWS_PER_TILE), cg])
            plsc.subcore_barrier()


def _msg_phase(hcflat, src1d, dst1d, p, rflat, bias):
    mesh = plsc.VectorSubcoreMesh(core_axis_name="c", subcore_axis_name="s",
                                  num_cores=NCORES, num_subcores=NSUB)
    f = pl.kernel(
        _msg_body,
        out_type=jax.ShapeDtypeStruct((N_PAD, NCHUNK, FC), _f32),
        mesh=mesh,
        scratch_types=[
            pltpu.VMEM_SHARED((N_PAD * H,), _f32),
            pltpu.VMEM_SHARED((N_PAD, FC), _f32),
            pltpu.VMEM((C_EDGES,), _f32),
            pltpu.VMEM((128,), _f32),
            pltpu.VMEM((128,), _i32),
            pltpu.VMEM((128,), _f32),
            pltpu.VMEM((128,), _i32),
            pltpu.VMEM((128,), _i32),
            pltpu.VMEM((128,), _i32),
            pltpu.VMEM((128,), _i32),
            pltpu.VMEM((128,), _i32),
            pltpu.VMEM((128,), _i32),
            pltpu.VMEM((128, FC), _f32),
            pltpu.VMEM((128, FC), _f32),
            pltpu.VMEM((FC,), _f32),
            pltpu.SemaphoreType.DMA,
            pltpu.SemaphoreType.DMA,
            pltpu.SemaphoreType.DMA,
            pltpu.SemaphoreType.DMA,
            pltpu.SemaphoreType.DMA,
        ],
    )
    return f(hcflat, src1d, dst1d, p, rflat, bias)


def kernel(x, edge_index, W, att_src, att_dst, bias):
    n = N
    wr = W.reshape(D, H, C)
    a_src = jnp.einsum("dhc,hc->dh", wr, att_src)
    a_dst = jnp.einsum("dhc,hc->dh", wr, att_dst)
    a8 = jnp.concatenate([a_src, a_dst], axis=1)
    a8p = jnp.pad(a8, ((0, 0), (0, 120)))

    x_pad = jnp.pad(x, ((0, N_PAD - n), (0, 0)))

    src = edge_index[0]
    dst = edge_index[1]
    loop = jnp.arange(n, dtype=src.dtype)
    pad_cnt = E_PAD - (src.shape[0] + n)
    pad_i = jnp.arange(pad_cnt, dtype=src.dtype)
    src_p = jnp.concatenate([src, loop, pad_i % n])
    dst_p = jnp.concatenate([dst, loop, n + (pad_i % (N_PAD - n))])

    wc = jnp.transpose(W.reshape(D, NCHUNK, FC), (1, 0, 2))
    hc = _matmul_chunks(x_pad, wc)
    a_full, amax = _logits(x_pad, a8p)
    asdf = a_full[:, :8].reshape(N_PAD * 8)
    k4 = (jnp.maximum(amax[0, :4], 0.0) + jnp.maximum(amax[0, 4:8], 0.0))
    k64 = jnp.repeat(k4, 16)

    p, denom2 = _edge_phase(src_p, dst_p, asdf, k64)
    rflat = 1.0 / (denom2[0] + denom2[1] + 1e-16)

    hcflat = hc.reshape(NCHUNK * N_PAD, FC)
    outc = _msg_phase(hcflat, src_p, dst_p, p, rflat, bias)

    out = outc.reshape(N_PAD, HC)[:n]
    return out
